# Initial kernel scaffold; baseline (speedup 1.0000x reference)
import functools

import jax
import jax.numpy as jnp
from jax import lax
from jax.experimental import pallas as pl
from jax.experimental.pallas import tpu as pltpu
from jax.experimental.pallas import tpu_sc as plsc

_D = 128
_B = 32
_S = 2048
_NSEG = 2

_NC = 2
_NS = 16
_NW = _NC * _NS
_SW = _S // _NW
_SS = 4
_NCH = _SW // _SS
_TCH = _SS * _B
_ND = _D // 16


def _rsqrt(v):
    y = lax.bitcast_convert_type(
        jnp.int32(0x5F3759DF) - (lax.bitcast_convert_type(v, jnp.int32) >> 1),
        jnp.float32)
    for _ in range(3):
        y = y * (1.5 - 0.5 * v * y * y)
    return y


def _body(x_hbm, seg_hbm, tok_hbm, pos_hbm, segt_hbm, gam_hbm, bet_hbm,
          out_hbm, idx_v, segi_v, pos_v, segt_v, gam_v, bet_v, tok_v, out_v,
          sem):
    wid = lax.axis_index("s") * _NC + lax.axis_index("c")
    tok0 = wid * (_SW * _B)
    s0 = wid * _SW

    pltpu.sync_copy(x_hbm.at[pl.ds(tok0, _SW * _B)], idx_v)
    pltpu.sync_copy(seg_hbm.at[pl.ds(tok0, _SW * _B)], segi_v)
    pltpu.sync_copy(pos_hbm.at[pl.ds(s0, _SW)], pos_v)
    pltpu.sync_copy(segt_hbm, segt_v)
    pltpu.sync_copy(gam_hbm, gam_v)
    pltpu.sync_copy(bet_hbm, bet_v)

    gam = [gam_v[pl.ds(dd * 16, 16)] for dd in range(_ND)]
    bet = [bet_v[pl.ds(dd * 16, 16)] for dd in range(_ND)]
    sg0 = [segt_v[0, pl.ds(dd * 16, 16)] for dd in range(_ND)]
    sg1 = [segt_v[1, pl.ds(dd * 16, 16)] for dd in range(_ND)]

    def chunk_body(c, carry):
        pltpu.async_copy(tok_hbm.at[idx_v.at[pl.ds(c * _TCH, _TCH)]],
                         tok_v, sem).wait()
        for s_loc in range(_SS):
            pos_row = [pos_v[c * _SS + s_loc, pl.ds(dd * 16, 16)]
                       for dd in range(_ND)]

            def tok_body(b, _):
                t = s_loc * _B + b
                sv = segi_v[c * _TCH + t]
                mseg = jnp.full((16,), sv, jnp.int32) == 0
                ssum = jnp.zeros((16,), jnp.float32)
                ssq = jnp.zeros((16,), jnp.float32)
                e = []
                for dd in range(_ND):
                    sd = jnp.where(mseg, sg0[dd], sg1[dd])
                    ed = tok_v[t, pl.ds(dd * 16, 16)] + pos_row[dd] + sd
                    e.append(ed)
                    ssum = ssum + ed
                    ssq = ssq + ed * ed
                mean = jnp.sum(ssum) * (1.0 / _D)
                var = jnp.sum(ssq) * (1.0 / _D) - mean * mean
                r = _rsqrt(jnp.full((16,), var + 1e-5, jnp.float32))
                mb = jnp.full((16,), mean, jnp.float32)
                for dd in range(_ND):
                    out_v[b, s_loc, pl.ds(dd * 16, 16)] = (
                        (e[dd] - mb) * r * gam[dd] + bet[dd])
                return _

            lax.fori_loop(0, _B, tok_body, 0)
        pltpu.sync_copy(out_v, out_hbm.at[:, pl.ds(s0 + c * _SS, _SS), :])
        return carry

    lax.fori_loop(0, _NCH, chunk_body, 0)


@jax.jit
def _emb_ln(x_flat, seg_flat, token_table, pos_table, seg_table, gamma, beta):
    mesh = plsc.VectorSubcoreMesh(core_axis_name="c", subcore_axis_name="s")
    return pl.kernel(
        _body,
        mesh=mesh,
        out_type=jax.ShapeDtypeStruct((_B, _S, _D), jnp.float32),
        scratch_types=[
            pltpu.VMEM((_SW * _B,), jnp.int32),
            pltpu.VMEM((_SW * _B,), jnp.int32),
            pltpu.VMEM((_SW, _D), jnp.float32),
            pltpu.VMEM((_NSEG, _D), jnp.float32),
            pltpu.VMEM((_D,), jnp.float32),
            pltpu.VMEM((_D,), jnp.float32),
            pltpu.VMEM((_TCH, _D), jnp.float32),
            pltpu.VMEM((_B, _SS, _D), jnp.float32),
            pltpu.SemaphoreType.DMA,
        ],
    )(x_flat, seg_flat, token_table, pos_table, seg_table, gamma, beta)


def kernel(x, seg, token_table, pos_table, seg_table, gamma, beta):
    x_flat = jnp.swapaxes(x, 0, 1).reshape(-1)
    seg_flat = jnp.swapaxes(seg, 0, 1).reshape(-1)
    return _emb_ln(x_flat, seg_flat, token_table, pos_table, seg_table,
                   gamma, beta)


# speedup vs baseline: 1.8934x; 1.8934x over previous
import functools

import jax
import jax.numpy as jnp
import numpy as np
from jax import lax
from jax.experimental import pallas as pl
from jax.experimental.pallas import tpu as pltpu
from jax.experimental.pallas import tpu_sc as plsc

_D = 128
_B = 32
_S = 2048
_NSEG = 2

_NC = 2
_NS = 16
_NW = _NC * _NS
_SW = _S // _NW
_SS = 4
_NCH = _SW // _SS
_TCH = _SS * _B
_ND = _D // 16


def _lane_sum(x):
    iota = lax.iota(jnp.int32, 16)
    for sh in (8, 4, 2, 1):
        x = x + x.at[iota ^ sh].get(mode="promise_in_bounds")
    return x


def _rsqrt(v):
    y = lax.bitcast_convert_type(
        jnp.int32(0x5F3759DF) - (lax.bitcast_convert_type(v, jnp.int32) >> 1),
        jnp.float32)
    for _ in range(3):
        y = y * (1.5 - 0.5 * v * y * y)
    return y


def _body(x_hbm, seg_hbm, tok_hbm, pos_hbm, segt_hbm, gam_hbm, bet_hbm,
          out_hbm, idx_v, segi_v, pos_v, segt_v, gam_v, bet_v, tok_v, out_v,
          sem):
    wid = lax.axis_index("s") * _NC + lax.axis_index("c")
    tok0 = wid * (_SW * _B)
    s0 = wid * _SW

    pltpu.sync_copy(x_hbm.at[pl.ds(tok0, _SW * _B)], idx_v)
    pltpu.sync_copy(seg_hbm.at[pl.ds(tok0, _SW * _B)], segi_v)
    pltpu.sync_copy(pos_hbm.at[pl.ds(s0, _SW)], pos_v)
    pltpu.sync_copy(segt_hbm, segt_v)
    pltpu.sync_copy(gam_hbm, gam_v)
    pltpu.sync_copy(bet_hbm, bet_v)

    gam = [gam_v[pl.ds(dd * 16, 16)] for dd in range(_ND)]
    bet = [bet_v[pl.ds(dd * 16, 16)] for dd in range(_ND)]
    sg0 = [segt_v[0, pl.ds(dd * 16, 16)] for dd in range(_ND)]
    dsg = [segt_v[1, pl.ds(dd * 16, 16)] - sg0[dd] for dd in range(_ND)]

    def chunk_body(c, carry):
        pltpu.async_copy(tok_hbm.at[idx_v.at[pl.ds(c * _TCH, _TCH)]],
                         tok_v, sem).wait()
        for s_loc in range(_SS):
            pos_row = [pos_v[c * _SS + s_loc, pl.ds(dd * 16, 16)]
                       for dd in range(_ND)]
            for half in range(_B // 16):
                svf = segi_v[pl.ds(c * _TCH + s_loc * _B + half * 16, 16)
                             ].astype(jnp.float32)

                def tok_body(b2, _):
                    t = s_loc * _B + half * 16 + b2
                    segf = svf.at[jnp.full((16,), b2, jnp.int32)].get(
                        mode="promise_in_bounds")
                    ssum = jnp.zeros((16,), jnp.float32)
                    ssq = jnp.zeros((16,), jnp.float32)
                    e = []
                    for dd in range(_ND):
                        ed = (tok_v[t, pl.ds(dd * 16, 16)] + pos_row[dd]
                              + (sg0[dd] + segf * dsg[dd]))
                        e.append(ed)
                        ssum = ssum + ed
                        ssq = ssq + ed * ed
                    tot = _lane_sum(ssum)
                    tot2 = _lane_sum(ssq)
                    mean = tot * (1.0 / _D)
                    var = tot2 * (1.0 / _D) - mean * mean
                    r = _rsqrt(var + 1e-5)
                    for dd in range(_ND):
                        out_v[half * 16 + b2, s_loc, pl.ds(dd * 16, 16)] = (
                            (e[dd] - mean) * r * gam[dd] + bet[dd])
                    return _

                lax.fori_loop(0, 16, tok_body, 0)
        pltpu.sync_copy(out_v, out_hbm.at[:, pl.ds(s0 + c * _SS, _SS), :])
        return carry

    lax.fori_loop(0, _NCH, chunk_body, 0)


@jax.jit
def _emb_ln(x_flat, seg_flat, token_table, pos_table, seg_table, gamma, beta):
    mesh = plsc.VectorSubcoreMesh(core_axis_name="c", subcore_axis_name="s")
    return pl.kernel(
        _body,
        mesh=mesh,
        out_type=jax.ShapeDtypeStruct((_B, _S, _D), jnp.float32),
        scratch_types=[
            pltpu.VMEM((_SW * _B,), jnp.int32),
            pltpu.VMEM((_SW * _B,), jnp.int32),
            pltpu.VMEM((_SW, _D), jnp.float32),
            pltpu.VMEM((_NSEG, _D), jnp.float32),
            pltpu.VMEM((_D,), jnp.float32),
            pltpu.VMEM((_D,), jnp.float32),
            pltpu.VMEM((_TCH, _D), jnp.float32),
            pltpu.VMEM((_B, _SS, _D), jnp.float32),
            pltpu.SemaphoreType.DMA,
        ],
    )(x_flat, seg_flat, token_table, pos_table, seg_table, gamma, beta)


def kernel(x, seg, token_table, pos_table, seg_table, gamma, beta):
    x_flat = jnp.swapaxes(x, 0, 1).reshape(-1)
    seg_flat = jnp.swapaxes(seg, 0, 1).reshape(-1)
    return _emb_ln(x_flat, seg_flat, token_table, pos_table, seg_table,
                   gamma, beta)
